# baseline (speedup 1.0000x reference)
import functools

import jax
import jax.numpy as jnp
from jax import lax
from jax.experimental import pallas as pl
from jax.experimental.pallas import tpu as pltpu
from jax.experimental.pallas import tpu_sc as plsc

N = 10000
E = 160000
D = 256
HALF = 128
CHUNK = 128
NC, NS = 2, 16
CPS = 80
NCHUNK = CPS * NS
EP = NCHUNK * CHUNK
RPS = 632
RPS_LAST = N - RPS * (NS - 1)
NA = N + 8
F32 = jnp.float32


def _for_my_rows(s, fn):
  @pl.when(s < NS - 1)
  def _():
    fn(RPS)
  @pl.when(s == NS - 1)
  def _():
    fn(RPS_LAST)


IBLK = 16
NIB = CPS // IBLK
CROWS = 80


def _make_segsum(with_hist):
  mesh = plsc.VectorSubcoreMesh(core_axis_name="c", subcore_axis_name="s")
  out_type = [jax.ShapeDtypeStruct((N, HALF), F32),
              jax.ShapeDtypeStruct((N, HALF), F32)]
  scratch = [
      pltpu.VMEM((IBLK, CHUNK), jnp.int32),
      pltpu.VMEM((IBLK, CHUNK), jnp.int32),
      pltpu.VMEM_SHARED((NA, HALF), F32),
      pltpu.VMEM((CHUNK, HALF), F32),
      pltpu.VMEM((CHUNK, HALF), F32),
      pltpu.SemaphoreType.DMA,
      pltpu.SemaphoreType.DMA,
      pltpu.SemaphoreType.DMA,
      pltpu.SemaphoreType.DMA,
  ]
  if with_hist:
    out_type.append(jax.ShapeDtypeStruct((CROWS, CHUNK), F32))
    scratch += [
        pltpu.VMEM((CROWS, CHUNK), F32),
        pltpu.VMEM((CROWS,), jnp.int32),
        pltpu.VMEM_SHARED((CROWS, CHUNK), F32),
    ]

  def body(*refs):
    if with_hist:
      (tab_lo, tab_hi, src3d, dst3d, zrows, iota_h, out_lo, out_hi, out_cnt,
       src_v, dst_v, acc, rows0, rows1, sg0, sg1, ss0, ss1,
       hist_v, iota_v, cnt_sp) = refs
    else:
      (tab_lo, tab_hi, src3d, dst3d, zrows, out_lo, out_hi,
       src_v, dst_v, acc, rows0, rows1, sg0, sg1, ss0, ss1) = refs
    rows = (rows0, rows1)
    semg = (sg0, sg1)
    sems = (ss0, ss1)
    c = lax.axis_index("c")
    s = lax.axis_index("s")

    _for_my_rows(s, lambda r: pltpu.sync_copy(zrows.at[pl.ds(0, r)],
                                              acc.at[pl.ds(s * RPS, r)]))
    if with_hist:
      @pl.when(c == 0)
      def _():
        pltpu.sync_copy(zrows.at[pl.ds(0, CROWS)], hist_v)
        pltpu.sync_copy(iota_h, iota_v)
        @pl.when(s == 0)
        def _():
          pltpu.sync_copy(zrows.at[pl.ds(0, CROWS)], cnt_sp)
    plsc.subcore_barrier()

    one = jnp.ones((16,), F32)

    def run(tab, do_hist):
      def gather(j, b):
        pltpu.async_copy(tab.at[src_v.at[j]], rows[b], semg[b])

      def wait_gather(j, b):
        pltpu.make_async_copy(tab.at[src_v.at[j]], rows[b], semg[b]).wait()

      def scatter(j, b):
        pltpu.async_copy(rows[b], acc.at[dst_v.at[j]], sems[b], add=True)

      def wait_scatter(j, b):
        pltpu.make_async_copy(rows[b], acc.at[dst_v.at[j]], sems[b]).wait()

      def hist(j):
        for i in range(CHUNK // 16):
          ids = dst_v[j, pl.ds(i * 16, 16)]
          r = lax.shift_right_logical(ids, 7)
          col = lax.bitwise_and(ids, 127)
          plsc.addupdate_scatter(hist_v, [r, col], one)

      def block(ib, carry):
        pltpu.sync_copy(src3d.at[s, pl.ds(ib * IBLK, IBLK)], src_v)
        pltpu.sync_copy(dst3d.at[s, pl.ds(ib * IBLK, IBLK)], dst_v)
        gather(0, 0)
        for j in range(IBLK):
          b = j % 2
          wait_gather(j, b)
          scatter(j, b)
          if do_hist:
            hist(j)
          if j + 1 < IBLK:
            if j >= 1:
              wait_scatter(j - 1, 1 - b)
            gather(j + 1, 1 - b)
        wait_scatter(IBLK - 2, 0)
        wait_scatter(IBLK - 1, 1)
        return carry
      lax.fori_loop(0, NIB, block, 0)
      if do_hist:
        pltpu.sync_copy(hist_v, cnt_sp.at[iota_v], add=True)

    @pl.when(c == 0)
    def _():
      run(tab_lo, with_hist)

    @pl.when(c == 1)
    def _():
      run(tab_hi, False)

    plsc.subcore_barrier()

    @pl.when(c == 0)
    def _():
      _for_my_rows(s, lambda r: pltpu.sync_copy(acc.at[pl.ds(s * RPS, r)],
                                                out_lo.at[pl.ds(s * RPS, r)]))
      if with_hist:
        @pl.when(s == 0)
        def _():
          pltpu.sync_copy(cnt_sp, out_cnt)

    @pl.when(c == 1)
    def _():
      _for_my_rows(s, lambda r: pltpu.sync_copy(acc.at[pl.ds(s * RPS, r)],
                                                out_hi.at[pl.ds(s * RPS, r)]))

  return pl.kernel(body, out_type=out_type, mesh=mesh, scratch_types=scratch,
                   compiler_params=pltpu.CompilerParams(
                       needs_layout_passes=False))


_segsum_h = _make_segsum(True)
_segsum = _make_segsum(False)


ROWS = 1000
_P = lax.Precision.HIGHEST
_PH = lax.Precision.DEFAULT


def _dot(a, b):
  return jnp.dot(a, b, precision=_PH, preferred_element_type=F32)


def _root_body(xl, xh, wrt, wrb, b, r):
  r[...] = _dot(xl[...], wrt[...]) + _dot(xh[...], wrb[...]) + b[...]


def _layer_body(sl, sh, c0, wlt, wlb, r, ol, oh):
  inv = 1.0 / jnp.maximum(c0[...], 1.0)
  acc = _dot(sl[...] * inv, wlt[...]) + _dot(sh[...] * inv, wlb[...]) + r[...]
  h = jnp.maximum(acc, 0.0)
  ol[...] = h[:, :HALF]
  oh[...] = h[:, HALF:]


def _head_body(sl, sh, c0, wlt, wlb, r, wo, bo, o):
  inv = 1.0 / jnp.maximum(c0[...], 1.0)
  acc = _dot(sl[...] * inv, wlt[...]) + _dot(sh[...] * inv, wlb[...]) + r[...]
  h = jnp.maximum(acc, 0.0)
  o[...] = _dot(h, wo[...]) + bo[...]


def _row_spec(cols):
  return pl.BlockSpec((ROWS, cols), lambda i: (i, 0))


def _full_spec(r, c):
  return pl.BlockSpec((r, c), lambda i: (0, 0))


_common_in_specs = [
    _row_spec(HALF), _row_spec(HALF),
    _row_spec(1),
    _full_spec(HALF, D), _full_spec(HALF, D),
    _row_spec(D),
]

_tc_root = pl.pallas_call(
    _root_body,
    grid=(N // ROWS,),
    in_specs=[_row_spec(HALF), _row_spec(HALF),
              _full_spec(HALF, D), _full_spec(HALF, D), _full_spec(1, D)],
    out_specs=_row_spec(D),
    out_shape=jax.ShapeDtypeStruct((N, D), F32),
)

_tc_layer = pl.pallas_call(
    _layer_body,
    grid=(N // ROWS,),
    in_specs=_common_in_specs,
    out_specs=[_row_spec(HALF), _row_spec(HALF)],
    out_shape=[jax.ShapeDtypeStruct((N, HALF), F32),
               jax.ShapeDtypeStruct((N, HALF), F32)],
)

_tc_head = pl.pallas_call(
    _head_body,
    grid=(N // ROWS,),
    in_specs=_common_in_specs + [_full_spec(D, HALF), _full_spec(1, HALF)],
    out_specs=_row_spec(HALF),
    out_shape=jax.ShapeDtypeStruct((N, HALF), F32),
)


@jax.jit
def kernel(x, edge_index, W1l, b1, W1r, W2l, b2, W2r, Wo, bo):
  pad = EP - E
  src3d = jnp.concatenate(
      [edge_index[0], jnp.zeros((pad,), jnp.int32)]).reshape(NS, CPS, CHUNK)
  dst3d = jnp.concatenate(
      [edge_index[1], jnp.full((pad,), N, jnp.int32)]).reshape(NS, CPS, CHUNK)
  x_lo = x[:, :HALF]
  x_hi = x[:, HALF:]
  zrows = jnp.zeros((RPS, HALF), F32)
  iota_h = jnp.arange(CROWS, dtype=jnp.int32)

  r1 = _tc_root(x_lo, x_hi, W1r[:HALF], W1r[HALF:], b1.reshape(1, D))
  s1_lo, s1_hi, cnt80 = _segsum_h(x_lo, x_hi, src3d, dst3d, zrows, iota_h)
  cnt = cnt80.reshape(-1)[:N].reshape(N, 1)
  h1_lo, h1_hi = _tc_layer(s1_lo, s1_hi, cnt,
                           W1l[:HALF], W1l[HALF:], r1)
  r2 = _tc_root(h1_lo, h1_hi, W2r[:HALF], W2r[HALF:], b2.reshape(1, D))
  s2_lo, s2_hi = _segsum(h1_lo, h1_hi, src3d, dst3d, zrows)
  wo_pad = jnp.pad(Wo, ((0, 0), (0, HALF - 1)))
  bo_pad = jnp.pad(bo.reshape(1, 1), ((0, 0), (0, HALF - 1)))
  out = _tc_head(s2_lo, s2_hi, cnt,
                 W2l[:HALF], W2l[HALF:], r2, wo_pad, bo_pad)
  return out[:, 0]


# speedup vs baseline: 1.0083x; 1.0083x over previous
import functools

import jax
import jax.numpy as jnp
from jax import lax
from jax.experimental import pallas as pl
from jax.experimental.pallas import tpu as pltpu
from jax.experimental.pallas import tpu_sc as plsc

N = 10000
E = 160000
D = 256
HALF = 128
CHUNK = 128
NC, NS = 2, 16
CPS = 80
NCHUNK = CPS * NS
EP = NCHUNK * CHUNK
RPS = 632
RPS_LAST = N - RPS * (NS - 1)
NA = N + 8
F32 = jnp.float32


def _for_my_rows(s, fn):
  @pl.when(s < NS - 1)
  def _():
    fn(RPS)
  @pl.when(s == NS - 1)
  def _():
    fn(RPS_LAST)


CROWS = 80


def _make_segsum(with_hist):
  IBLK = 16 if with_hist else 40
  NIB = CPS // IBLK
  mesh = plsc.VectorSubcoreMesh(core_axis_name="c", subcore_axis_name="s")
  out_type = [jax.ShapeDtypeStruct((N, HALF), F32),
              jax.ShapeDtypeStruct((N, HALF), F32)]
  scratch = [
      pltpu.VMEM((IBLK, CHUNK), jnp.int32),
      pltpu.VMEM((IBLK, CHUNK), jnp.int32),
      pltpu.VMEM_SHARED((NA, HALF), F32),
      pltpu.VMEM((CHUNK, HALF), F32),
      pltpu.VMEM((CHUNK, HALF), F32),
      pltpu.SemaphoreType.DMA,
      pltpu.SemaphoreType.DMA,
      pltpu.SemaphoreType.DMA,
      pltpu.SemaphoreType.DMA,
  ]
  if with_hist:
    out_type.append(jax.ShapeDtypeStruct((CROWS, CHUNK), F32))
    scratch += [
        pltpu.VMEM((CROWS, CHUNK), F32),
        pltpu.VMEM((CROWS,), jnp.int32),
        pltpu.VMEM_SHARED((CROWS, CHUNK), F32),
    ]

  def body(*refs):
    if with_hist:
      (tab_lo, tab_hi, src3d, dst3d, zrows, iota_h, out_lo, out_hi, out_cnt,
       src_v, dst_v, acc, rows0, rows1, sg0, sg1, ss0, ss1,
       hist_v, iota_v, cnt_sp) = refs
    else:
      (tab_lo, tab_hi, src3d, dst3d, zrows, out_lo, out_hi,
       src_v, dst_v, acc, rows0, rows1, sg0, sg1, ss0, ss1) = refs
    rows = (rows0, rows1)
    semg = (sg0, sg1)
    sems = (ss0, ss1)
    c = lax.axis_index("c")
    s = lax.axis_index("s")

    _for_my_rows(s, lambda r: pltpu.sync_copy(zrows.at[pl.ds(0, r)],
                                              acc.at[pl.ds(s * RPS, r)]))
    if with_hist:
      @pl.when(c == 0)
      def _():
        pltpu.sync_copy(zrows.at[pl.ds(0, CROWS)], hist_v)
        pltpu.sync_copy(iota_h, iota_v)
        @pl.when(s == 0)
        def _():
          pltpu.sync_copy(zrows.at[pl.ds(0, CROWS)], cnt_sp)
    plsc.subcore_barrier()

    one = jnp.ones((16,), F32)

    def run(tab, do_hist):
      def gather(j, b):
        pltpu.async_copy(tab.at[src_v.at[j]], rows[b], semg[b])

      def wait_gather(j, b):
        pltpu.make_async_copy(tab.at[src_v.at[j]], rows[b], semg[b]).wait()

      def scatter(j, b):
        pltpu.async_copy(rows[b], acc.at[dst_v.at[j]], sems[b], add=True)

      def wait_scatter(j, b):
        pltpu.make_async_copy(rows[b], acc.at[dst_v.at[j]], sems[b]).wait()

      def hist(j):
        for i in range(CHUNK // 16):
          ids = dst_v[j, pl.ds(i * 16, 16)]
          r = lax.shift_right_logical(ids, 7)
          col = lax.bitwise_and(ids, 127)
          plsc.addupdate_scatter(hist_v, [r, col], one)

      def block(ib, carry):
        pltpu.sync_copy(src3d.at[s, pl.ds(ib * IBLK, IBLK)], src_v)
        pltpu.sync_copy(dst3d.at[s, pl.ds(ib * IBLK, IBLK)], dst_v)
        gather(0, 0)
        for j in range(IBLK):
          b = j % 2
          wait_gather(j, b)
          scatter(j, b)
          if do_hist:
            hist(j)
          if j + 1 < IBLK:
            if j >= 1:
              wait_scatter(j - 1, 1 - b)
            gather(j + 1, 1 - b)
        wait_scatter(IBLK - 2, 0)
        wait_scatter(IBLK - 1, 1)
        return carry
      lax.fori_loop(0, NIB, block, 0)
      if do_hist:
        pltpu.sync_copy(hist_v, cnt_sp.at[iota_v], add=True)

    @pl.when(c == 0)
    def _():
      run(tab_lo, with_hist)

    @pl.when(c == 1)
    def _():
      run(tab_hi, False)

    plsc.subcore_barrier()

    @pl.when(c == 0)
    def _():
      _for_my_rows(s, lambda r: pltpu.sync_copy(acc.at[pl.ds(s * RPS, r)],
                                                out_lo.at[pl.ds(s * RPS, r)]))
      if with_hist:
        @pl.when(s == 0)
        def _():
          pltpu.sync_copy(cnt_sp, out_cnt)

    @pl.when(c == 1)
    def _():
      _for_my_rows(s, lambda r: pltpu.sync_copy(acc.at[pl.ds(s * RPS, r)],
                                                out_hi.at[pl.ds(s * RPS, r)]))

  return pl.kernel(body, out_type=out_type, mesh=mesh, scratch_types=scratch,
                   compiler_params=pltpu.CompilerParams(
                       needs_layout_passes=False))


_segsum_h = _make_segsum(True)
_segsum = _make_segsum(False)


ROWS = 1000
_P = lax.Precision.HIGHEST
_PH = lax.Precision.DEFAULT


def _dot(a, b):
  return jnp.dot(a, b, precision=_PH, preferred_element_type=F32)


def _root_body(xl, xh, wrt, wrb, b, r):
  r[...] = _dot(xl[...], wrt[...]) + _dot(xh[...], wrb[...]) + b[...]


def _layer_body(sl, sh, c0, wlt, wlb, r, ol, oh):
  inv = 1.0 / jnp.maximum(c0[...], 1.0)
  acc = _dot(sl[...] * inv, wlt[...]) + _dot(sh[...] * inv, wlb[...]) + r[...]
  h = jnp.maximum(acc, 0.0)
  ol[...] = h[:, :HALF]
  oh[...] = h[:, HALF:]


def _head_body(sl, sh, c0, wlt, wlb, r, wo, bo, o):
  inv = 1.0 / jnp.maximum(c0[...], 1.0)
  acc = _dot(sl[...] * inv, wlt[...]) + _dot(sh[...] * inv, wlb[...]) + r[...]
  h = jnp.maximum(acc, 0.0)
  o[...] = _dot(h, wo[...]) + bo[...]


def _row_spec(cols):
  return pl.BlockSpec((ROWS, cols), lambda i: (i, 0))


def _full_spec(r, c):
  return pl.BlockSpec((r, c), lambda i: (0, 0))


_common_in_specs = [
    _row_spec(HALF), _row_spec(HALF),
    _row_spec(1),
    _full_spec(HALF, D), _full_spec(HALF, D),
    _row_spec(D),
]

_tc_root = pl.pallas_call(
    _root_body,
    grid=(N // ROWS,),
    in_specs=[_row_spec(HALF), _row_spec(HALF),
              _full_spec(HALF, D), _full_spec(HALF, D), _full_spec(1, D)],
    out_specs=_row_spec(D),
    out_shape=jax.ShapeDtypeStruct((N, D), F32),
)

_tc_layer = pl.pallas_call(
    _layer_body,
    grid=(N // ROWS,),
    in_specs=_common_in_specs,
    out_specs=[_row_spec(HALF), _row_spec(HALF)],
    out_shape=[jax.ShapeDtypeStruct((N, HALF), F32),
               jax.ShapeDtypeStruct((N, HALF), F32)],
)

_tc_head = pl.pallas_call(
    _head_body,
    grid=(N // ROWS,),
    in_specs=_common_in_specs + [_full_spec(D, HALF), _full_spec(1, HALF)],
    out_specs=_row_spec(HALF),
    out_shape=jax.ShapeDtypeStruct((N, HALF), F32),
)


@jax.jit
def kernel(x, edge_index, W1l, b1, W1r, W2l, b2, W2r, Wo, bo):
  pad = EP - E
  src3d = jnp.concatenate(
      [edge_index[0], jnp.zeros((pad,), jnp.int32)]).reshape(NS, CPS, CHUNK)
  dst3d = jnp.concatenate(
      [edge_index[1], jnp.full((pad,), N, jnp.int32)]).reshape(NS, CPS, CHUNK)
  x_lo = x[:, :HALF]
  x_hi = x[:, HALF:]
  zrows = jnp.zeros((RPS, HALF), F32)
  iota_h = jnp.arange(CROWS, dtype=jnp.int32)

  r1 = _tc_root(x_lo, x_hi, W1r[:HALF], W1r[HALF:], b1.reshape(1, D))
  s1_lo, s1_hi, cnt80 = _segsum_h(x_lo, x_hi, src3d, dst3d, zrows, iota_h)
  cnt = cnt80.reshape(-1)[:N].reshape(N, 1)
  h1_lo, h1_hi = _tc_layer(s1_lo, s1_hi, cnt,
                           W1l[:HALF], W1l[HALF:], r1)
  r2 = _tc_root(h1_lo, h1_hi, W2r[:HALF], W2r[HALF:], b2.reshape(1, D))
  s2_lo, s2_hi = _segsum(h1_lo, h1_hi, src3d, dst3d, zrows)
  wo_pad = jnp.pad(Wo, ((0, 0), (0, HALF - 1)))
  bo_pad = jnp.pad(bo.reshape(1, 1), ((0, 0), (0, HALF - 1)))
  out = _tc_head(s2_lo, s2_hi, cnt,
                 W2l[:HALF], W2l[HALF:], r2, wo_pad, bo_pad)
  return out[:, 0]


# speedup vs baseline: 1.0299x; 1.0214x over previous
import functools

import jax
import jax.numpy as jnp
from jax import lax
from jax.experimental import pallas as pl
from jax.experimental.pallas import tpu as pltpu
from jax.experimental.pallas import tpu_sc as plsc

N = 10000
E = 160000
D = 256
HALF = 128
CHUNK = 128
NC, NS = 2, 16
CPS = 80
NCHUNK = CPS * NS
EP = NCHUNK * CHUNK
RPS = 632
RPS_LAST = N - RPS * (NS - 1)
NA = N + 8
F32 = jnp.float32


def _for_my_rows(s, fn):
  @pl.when(s < NS - 1)
  def _():
    fn(RPS)
  @pl.when(s == NS - 1)
  def _():
    fn(RPS_LAST)


IBLK = 40
NIB = CPS // IBLK


def _make_segsum():
  mesh = plsc.VectorSubcoreMesh(core_axis_name="c", subcore_axis_name="s")
  out_type = [jax.ShapeDtypeStruct((N, HALF), F32),
              jax.ShapeDtypeStruct((N, HALF), F32)]
  scratch = [
      pltpu.VMEM((IBLK, CHUNK), jnp.int32),
      pltpu.VMEM((IBLK, CHUNK), jnp.int32),
      pltpu.VMEM_SHARED((NA, HALF), F32),
      pltpu.VMEM((CHUNK, HALF), F32),
      pltpu.VMEM((CHUNK, HALF), F32),
      pltpu.SemaphoreType.DMA,
      pltpu.SemaphoreType.DMA,
      pltpu.SemaphoreType.DMA,
      pltpu.SemaphoreType.DMA,
  ]

  def body(tab_lo, tab_hi, src3d, dst3d, zrows, out_lo, out_hi,
           src_v, dst_v, acc, rows0, rows1, sg0, sg1, ss0, ss1):
    rows = (rows0, rows1)
    semg = (sg0, sg1)
    sems = (ss0, ss1)
    c = lax.axis_index("c")
    s = lax.axis_index("s")

    _for_my_rows(s, lambda r: pltpu.sync_copy(zrows.at[pl.ds(0, r)],
                                              acc.at[pl.ds(s * RPS, r)]))
    plsc.subcore_barrier()

    def run(tab):
      GH = CHUNK // 2

      def gather(j, b):
        pltpu.async_copy(tab.at[src_v.at[j, pl.ds(0, GH)]],
                         rows[b].at[pl.ds(0, GH)], semg[b])
        pltpu.async_copy(tab.at[src_v.at[j, pl.ds(GH, GH)]],
                         rows[b].at[pl.ds(GH, GH)], semg[b])

      def wait_gather(j, b):
        pltpu.make_async_copy(tab.at[src_v.at[j]], rows[b], semg[b]).wait()

      def scatter(j, b):
        pltpu.async_copy(rows[b], acc.at[dst_v.at[j]], sems[b], add=True)

      def wait_scatter(j, b):
        pltpu.make_async_copy(rows[b], acc.at[dst_v.at[j]], sems[b]).wait()

      def block(ib, carry):
        pltpu.sync_copy(src3d.at[s, pl.ds(ib * IBLK, IBLK)], src_v)
        pltpu.sync_copy(dst3d.at[s, pl.ds(ib * IBLK, IBLK)], dst_v)
        gather(0, 0)
        for j in range(IBLK):
          b = j % 2
          wait_gather(j, b)
          scatter(j, b)
          if j + 1 < IBLK:
            if j >= 1:
              wait_scatter(j - 1, 1 - b)
            gather(j + 1, 1 - b)
        wait_scatter(IBLK - 2, 0)
        wait_scatter(IBLK - 1, 1)
        return carry
      lax.fori_loop(0, NIB, block, 0)

    @pl.when(c == 0)
    def _():
      run(tab_lo)

    @pl.when(c == 1)
    def _():
      run(tab_hi)

    plsc.subcore_barrier()

    @pl.when(c == 0)
    def _():
      _for_my_rows(s, lambda r: pltpu.sync_copy(acc.at[pl.ds(s * RPS, r)],
                                                out_lo.at[pl.ds(s * RPS, r)]))

    @pl.when(c == 1)
    def _():
      _for_my_rows(s, lambda r: pltpu.sync_copy(acc.at[pl.ds(s * RPS, r)],
                                                out_hi.at[pl.ds(s * RPS, r)]))

  return pl.kernel(body, out_type=out_type, mesh=mesh, scratch_types=scratch)


CROWS = 80


def _make_count():
  mesh = plsc.VectorSubcoreMesh(core_axis_name="c", subcore_axis_name="s")
  out_type = [jax.ShapeDtypeStruct((CROWS, CHUNK), F32),
              jax.ShapeDtypeStruct((CROWS, CHUNK), F32)]
  khalf = CPS // 2
  scratch = [
      pltpu.VMEM((CPS, CHUNK), jnp.int32),
      pltpu.VMEM((CROWS, CHUNK), F32),
      pltpu.VMEM((CROWS,), jnp.int32),
      pltpu.VMEM_SHARED((CROWS, CHUNK), F32),
  ]

  def body(dst3d, iota_h, zrows, out0, out1, dst_v, hist_v, iota_v, cnt_sp):
    c = lax.axis_index("c")
    s = lax.axis_index("s")

    pltpu.sync_copy(dst3d.at[s], dst_v)
    pltpu.sync_copy(zrows.at[pl.ds(0, CROWS)], hist_v)
    pltpu.sync_copy(iota_h, iota_v)
    @pl.when(s == 0)
    def _():
      pltpu.sync_copy(zrows.at[pl.ds(0, CROWS)], cnt_sp)
    plsc.subcore_barrier()

    base = jnp.where(c == 0, 0, khalf)
    one = jnp.ones((16,), F32)

    def blk(j, carry):
      row = base + j
      for i in range(CHUNK // 16):
        ids = dst_v[row, pl.ds(i * 16, 16)]
        r = lax.shift_right_logical(ids, 7)
        col = lax.bitwise_and(ids, 127)
        plsc.addupdate_scatter(hist_v, [r, col], one)
      return carry
    lax.fori_loop(0, khalf, blk, 0)

    pltpu.sync_copy(hist_v, cnt_sp.at[iota_v], add=True)
    plsc.subcore_barrier()

    @pl.when(s == 0)
    def _():
      @pl.when(c == 0)
      def _():
        pltpu.sync_copy(cnt_sp, out0)
      @pl.when(c == 1)
      def _():
        pltpu.sync_copy(cnt_sp, out1)

  return pl.kernel(body, out_type=out_type, mesh=mesh, scratch_types=scratch,
                   compiler_params=pltpu.CompilerParams(
                       needs_layout_passes=False))


_segsum = _make_segsum()
_count = _make_count()


ROWS = 1000
_P = lax.Precision.HIGHEST
_PH = lax.Precision.DEFAULT


def _dot(a, b):
  return jnp.dot(a, b, precision=_PH, preferred_element_type=F32)


def _root_body(xl, xh, wrt, wrb, b, r):
  r[...] = _dot(xl[...], wrt[...]) + _dot(xh[...], wrb[...]) + b[...]


def _layer_body(sl, sh, c0, c1, wlt, wlb, r, ol, oh):
  inv = 1.0 / jnp.maximum(c0[...] + c1[...], 1.0)
  acc = _dot(sl[...] * inv, wlt[...]) + _dot(sh[...] * inv, wlb[...]) + r[...]
  h = jnp.maximum(acc, 0.0)
  ol[...] = h[:, :HALF]
  oh[...] = h[:, HALF:]


def _head_body(sl, sh, c0, c1, wlt, wlb, r, wo, bo, o):
  inv = 1.0 / jnp.maximum(c0[...] + c1[...], 1.0)
  acc = _dot(sl[...] * inv, wlt[...]) + _dot(sh[...] * inv, wlb[...]) + r[...]
  h = jnp.maximum(acc, 0.0)
  o[...] = _dot(h, wo[...]) + bo[...]


def _row_spec(cols):
  return pl.BlockSpec((ROWS, cols), lambda i: (i, 0))


def _full_spec(r, c):
  return pl.BlockSpec((r, c), lambda i: (0, 0))


_common_in_specs = [
    _row_spec(HALF), _row_spec(HALF),
    _row_spec(1), _row_spec(1),
    _full_spec(HALF, D), _full_spec(HALF, D),
    _row_spec(D),
]

_tc_root = pl.pallas_call(
    _root_body,
    grid=(N // ROWS,),
    in_specs=[_row_spec(HALF), _row_spec(HALF),
              _full_spec(HALF, D), _full_spec(HALF, D), _full_spec(1, D)],
    out_specs=_row_spec(D),
    out_shape=jax.ShapeDtypeStruct((N, D), F32),
)

_tc_layer = pl.pallas_call(
    _layer_body,
    grid=(N // ROWS,),
    in_specs=_common_in_specs,
    out_specs=[_row_spec(HALF), _row_spec(HALF)],
    out_shape=[jax.ShapeDtypeStruct((N, HALF), F32),
               jax.ShapeDtypeStruct((N, HALF), F32)],
)

_tc_head = pl.pallas_call(
    _head_body,
    grid=(N // ROWS,),
    in_specs=_common_in_specs + [_full_spec(D, HALF), _full_spec(1, HALF)],
    out_specs=_row_spec(HALF),
    out_shape=jax.ShapeDtypeStruct((N, HALF), F32),
)


@jax.jit
def kernel(x, edge_index, W1l, b1, W1r, W2l, b2, W2r, Wo, bo):
  pad = EP - E
  src3d = jnp.concatenate(
      [edge_index[0], jnp.zeros((pad,), jnp.int32)]).reshape(NS, CPS, CHUNK)
  dst3d = jnp.concatenate(
      [edge_index[1], jnp.full((pad,), N, jnp.int32)]).reshape(NS, CPS, CHUNK)
  x_lo = x[:, :HALF]
  x_hi = x[:, HALF:]
  zrows = jnp.zeros((RPS, HALF), F32)
  iota_h = jnp.arange(CROWS, dtype=jnp.int32)

  cnt0, cnt1 = _count(dst3d, iota_h, zrows)
  cnt0 = cnt0.reshape(-1)[:N].reshape(N, 1)
  cnt1 = cnt1.reshape(-1)[:N].reshape(N, 1)
  r1 = _tc_root(x_lo, x_hi, W1r[:HALF], W1r[HALF:], b1.reshape(1, D))
  s1_lo, s1_hi = _segsum(x_lo, x_hi, src3d, dst3d, zrows)
  h1_lo, h1_hi = _tc_layer(s1_lo, s1_hi, cnt0, cnt1,
                           W1l[:HALF], W1l[HALF:], r1)
  r2 = _tc_root(h1_lo, h1_hi, W2r[:HALF], W2r[HALF:], b2.reshape(1, D))
  s2_lo, s2_hi = _segsum(h1_lo, h1_hi, src3d, dst3d, zrows)
  wo_pad = jnp.pad(Wo, ((0, 0), (0, HALF - 1)))
  bo_pad = jnp.pad(bo.reshape(1, 1), ((0, 0), (0, HALF - 1)))
  out = _tc_head(s2_lo, s2_hi, cnt0, cnt1,
                 W2l[:HALF], W2l[HALF:], r2, wo_pad, bo_pad)
  return out[:, 0]


# speedup vs baseline: 1.0300x; 1.0001x over previous
import functools

import jax
import jax.numpy as jnp
from jax import lax
from jax.experimental import pallas as pl
from jax.experimental.pallas import tpu as pltpu
from jax.experimental.pallas import tpu_sc as plsc

N = 10000
E = 160000
D = 256
HALF = 128
CHUNK = 128
NC, NS = 2, 16
CPS = 80
NCHUNK = CPS * NS
EP = NCHUNK * CHUNK
RPS = 632
RPS_LAST = N - RPS * (NS - 1)
NA = N + 8
F32 = jnp.float32


def _for_my_rows(s, fn):
  @pl.when(s < NS - 1)
  def _():
    fn(RPS)
  @pl.when(s == NS - 1)
  def _():
    fn(RPS_LAST)


IBLK = 40
NIB = CPS // IBLK


def _make_segsum():
  mesh = plsc.VectorSubcoreMesh(core_axis_name="c", subcore_axis_name="s")
  out_type = [jax.ShapeDtypeStruct((N, HALF), F32),
              jax.ShapeDtypeStruct((N, HALF), F32)]
  scratch = [
      pltpu.VMEM((IBLK, CHUNK), jnp.int32),
      pltpu.VMEM((IBLK, CHUNK), jnp.int32),
      pltpu.VMEM_SHARED((NA, HALF), F32),
      pltpu.VMEM((CHUNK, HALF), F32),
      pltpu.VMEM((CHUNK, HALF), F32),
      pltpu.SemaphoreType.DMA,
      pltpu.SemaphoreType.DMA,
      pltpu.SemaphoreType.DMA,
      pltpu.SemaphoreType.DMA,
  ]

  def body(tab_lo, tab_hi, src3d, dst3d, zrows, out_lo, out_hi,
           src_v, dst_v, acc, rows0, rows1, sg0, sg1, ss0, ss1):
    rows = (rows0, rows1)
    semg = (sg0, sg1)
    sems = (ss0, ss1)
    c = lax.axis_index("c")
    s = lax.axis_index("s")

    _for_my_rows(s, lambda r: pltpu.sync_copy(zrows.at[pl.ds(0, r)],
                                              acc.at[pl.ds(s * RPS, r)]))
    plsc.subcore_barrier()

    def run(tab):
      GH = CHUNK // 2

      def gather(j, b):
        pltpu.async_copy(tab.at[src_v.at[j, pl.ds(0, GH)]],
                         rows[b].at[pl.ds(0, GH)], semg[b])
        pltpu.async_copy(tab.at[src_v.at[j, pl.ds(GH, GH)]],
                         rows[b].at[pl.ds(GH, GH)], semg[b])

      def wait_gather(j, b):
        pltpu.make_async_copy(tab.at[src_v.at[j]], rows[b], semg[b]).wait()

      def scatter(j, b):
        pltpu.async_copy(rows[b], acc.at[dst_v.at[j]], sems[b], add=True)

      def wait_scatter(j, b):
        pltpu.make_async_copy(rows[b], acc.at[dst_v.at[j]], sems[b]).wait()

      def block(ib, carry):
        pltpu.sync_copy(src3d.at[s, pl.ds(ib * IBLK, IBLK)], src_v)
        pltpu.sync_copy(dst3d.at[s, pl.ds(ib * IBLK, IBLK)], dst_v)
        gather(0, 0)
        for j in range(IBLK):
          b = j % 2
          wait_gather(j, b)
          scatter(j, b)
          if j + 1 < IBLK:
            if j >= 1:
              wait_scatter(j - 1, 1 - b)
            gather(j + 1, 1 - b)
        wait_scatter(IBLK - 2, 0)
        wait_scatter(IBLK - 1, 1)
        return carry
      lax.fori_loop(0, NIB, block, 0)

    @pl.when(c == 0)
    def _():
      run(tab_lo)

    @pl.when(c == 1)
    def _():
      run(tab_hi)

    plsc.subcore_barrier()

    @pl.when(c == 0)
    def _():
      _for_my_rows(s, lambda r: pltpu.sync_copy(acc.at[pl.ds(s * RPS, r)],
                                                out_lo.at[pl.ds(s * RPS, r)]))

    @pl.when(c == 1)
    def _():
      _for_my_rows(s, lambda r: pltpu.sync_copy(acc.at[pl.ds(s * RPS, r)],
                                                out_hi.at[pl.ds(s * RPS, r)]))

  return pl.kernel(body, out_type=out_type, mesh=mesh, scratch_types=scratch)


CROWS = 80


def _make_count():
  mesh = plsc.VectorSubcoreMesh(core_axis_name="c", subcore_axis_name="s")
  out_type = [jax.ShapeDtypeStruct((CROWS, CHUNK), F32),
              jax.ShapeDtypeStruct((CROWS, CHUNK), F32)]
  khalf = CPS // 2
  scratch = [
      pltpu.VMEM((CPS, CHUNK), jnp.int32),
      pltpu.VMEM((CROWS, CHUNK), F32),
      pltpu.VMEM((CROWS,), jnp.int32),
      pltpu.VMEM_SHARED((CROWS, CHUNK), F32),
  ]

  def body(dst3d, iota_h, zrows, out0, out1, dst_v, hist_v, iota_v, cnt_sp):
    c = lax.axis_index("c")
    s = lax.axis_index("s")

    pltpu.sync_copy(dst3d.at[s], dst_v)
    pltpu.sync_copy(zrows.at[pl.ds(0, CROWS)], hist_v)
    pltpu.sync_copy(iota_h, iota_v)
    @pl.when(s == 0)
    def _():
      pltpu.sync_copy(zrows.at[pl.ds(0, CROWS)], cnt_sp)
    plsc.subcore_barrier()

    base = jnp.where(c == 0, 0, khalf)
    one = jnp.ones((16,), F32)

    def blk(j, carry):
      row = base + j
      for i in range(CHUNK // 16):
        ids = dst_v[row, pl.ds(i * 16, 16)]
        r = lax.shift_right_logical(ids, 7)
        col = lax.bitwise_and(ids, 127)
        plsc.addupdate_scatter(hist_v, [r, col], one)
      return carry
    lax.fori_loop(0, khalf, blk, 0)

    pltpu.sync_copy(hist_v, cnt_sp.at[iota_v], add=True)
    plsc.subcore_barrier()

    @pl.when(s == 0)
    def _():
      @pl.when(c == 0)
      def _():
        pltpu.sync_copy(cnt_sp, out0)
      @pl.when(c == 1)
      def _():
        pltpu.sync_copy(cnt_sp, out1)

  return pl.kernel(body, out_type=out_type, mesh=mesh, scratch_types=scratch,
                   compiler_params=pltpu.CompilerParams(
                       needs_layout_passes=False))


_segsum = _make_segsum()
_count = _make_count()


ROWS = 1000
_PH = lax.Precision.DEFAULT


def _dot(a, b):
  return jnp.dot(a, b, precision=_PH, preferred_element_type=F32)


def _root_body(xl, xh, wrt, wrb, b, r):
  r[...] = _dot(xl[...], wrt[...]) + _dot(xh[...], wrb[...]) + b[...]


def _layer_body(sl, sh, c0, c1, wlt, wlb, r, ol, oh):
  inv = 1.0 / jnp.maximum(c0[...] + c1[...], 1.0)
  acc = _dot(sl[...] * inv, wlt[...]) + _dot(sh[...] * inv, wlb[...]) + r[...]
  h = jnp.maximum(acc, 0.0)
  ol[...] = h[:, :HALF]
  oh[...] = h[:, HALF:]


def _head_body(sl, sh, c0, c1, wlt, wlb, r, wo, bo, o):
  inv = 1.0 / jnp.maximum(c0[...] + c1[...], 1.0)
  acc = _dot(sl[...] * inv, wlt[...]) + _dot(sh[...] * inv, wlb[...]) + r[...]
  h = jnp.maximum(acc, 0.0)
  o[...] = _dot(h, wo[...]) + bo[...]


def _row_spec(cols):
  return pl.BlockSpec((ROWS, cols), lambda i: (i, 0))


def _full_spec(r, c):
  return pl.BlockSpec((r, c), lambda i: (0, 0))


_common_in_specs = [
    _row_spec(HALF), _row_spec(HALF),
    _row_spec(1), _row_spec(1),
    _full_spec(HALF, D), _full_spec(HALF, D),
    _row_spec(D),
]

_tc_root = pl.pallas_call(
    _root_body,
    grid=(N // ROWS,),
    in_specs=[_row_spec(HALF), _row_spec(HALF),
              _full_spec(HALF, D), _full_spec(HALF, D), _full_spec(1, D)],
    out_specs=_row_spec(D),
    out_shape=jax.ShapeDtypeStruct((N, D), F32),
)

_tc_layer = pl.pallas_call(
    _layer_body,
    grid=(N // ROWS,),
    in_specs=_common_in_specs,
    out_specs=[_row_spec(HALF), _row_spec(HALF)],
    out_shape=[jax.ShapeDtypeStruct((N, HALF), F32),
               jax.ShapeDtypeStruct((N, HALF), F32)],
)

_tc_head = pl.pallas_call(
    _head_body,
    grid=(N // ROWS,),
    in_specs=_common_in_specs + [_full_spec(D, HALF), _full_spec(1, HALF)],
    out_specs=_row_spec(HALF),
    out_shape=jax.ShapeDtypeStruct((N, HALF), F32),
)


@jax.jit
def kernel(x, edge_index, W1l, b1, W1r, W2l, b2, W2r, Wo, bo):
  pad = EP - E
  src3d = jnp.concatenate(
      [edge_index[0], jnp.zeros((pad,), jnp.int32)]).reshape(NS, CPS, CHUNK)
  dst3d = jnp.concatenate(
      [edge_index[1], jnp.full((pad,), N, jnp.int32)]).reshape(NS, CPS, CHUNK)
  x_lo = x[:, :HALF]
  x_hi = x[:, HALF:]
  zrows = jnp.zeros((RPS, HALF), F32)
  iota_h = jnp.arange(CROWS, dtype=jnp.int32)

  cnt0, cnt1 = _count(dst3d, iota_h, zrows)
  cnt0 = cnt0.reshape(-1)[:N].reshape(N, 1)
  cnt1 = cnt1.reshape(-1)[:N].reshape(N, 1)
  r1 = _tc_root(x_lo, x_hi, W1r[:HALF], W1r[HALF:], b1.reshape(1, D))
  s1_lo, s1_hi = _segsum(x_lo, x_hi, src3d, dst3d, zrows)
  h1_lo, h1_hi = _tc_layer(s1_lo, s1_hi, cnt0, cnt1,
                           W1l[:HALF], W1l[HALF:], r1)
  r2 = _tc_root(h1_lo, h1_hi, W2r[:HALF], W2r[HALF:], b2.reshape(1, D))
  s2_lo, s2_hi = _segsum(h1_lo, h1_hi, src3d, dst3d, zrows)
  wo_pad = jnp.pad(Wo, ((0, 0), (0, HALF - 1)))
  bo_pad = jnp.pad(bo.reshape(1, 1), ((0, 0), (0, HALF - 1)))
  out = _tc_head(s2_lo, s2_hi, cnt0, cnt1,
                 W2l[:HALF], W2l[HALF:], r2, wo_pad, bo_pad)
  return out[:, 0]
